# baseline (speedup 1.0000x reference)
import functools
import jax
import jax.numpy as jnp
import numpy as np
from jax import lax
from jax.experimental import pallas as pl
from jax.experimental.pallas import tpu as pltpu
from jax.experimental.pallas import tpu_sc as plsc

N_NODES = 2048
N_EDGES = 32768
F = 32
BS = 4
MSG_STEPS = 20
E_TOT = N_EDGES + N_NODES
E_CHUNK = 1024
N_ECHUNK = E_TOT // E_CHUNK


E_PER_SUB = E_TOT // 16
E_VREGS = E_PER_SUB // 16
IDX_ROWS = E_PER_SUB // 128
ROWS_PER_PASS = 256
SC_ROW0 = 1024
SC_ROWS = N_NODES - SC_ROW0
N_PASSES = SC_ROWS // 2 // ROWS_PER_PASS
PASS_ELEMS = ROWS_PER_PASS * N_NODES
SUB_ELEMS = PASS_ELEMS // 16


def _sc_counts_body(dst_hbm, src_hbm, out_hbm, dstb, srcb, idxb, valb, zb,
                    shared):
    cid = lax.axis_index("c")
    sid = lax.axis_index("s")
    base = sid * E_PER_SUB
    pltpu.sync_copy(dst_hbm.at[pl.ds(base, E_PER_SUB)], dstb)
    pltpu.sync_copy(src_hbm.at[pl.ds(base, E_PER_SUB)], srcb)

    def zstep(i, carry):
        zb[pl.ds(i * 16, 16)] = jnp.zeros((16,), jnp.float32)
        return carry

    lax.fori_loop(0, SUB_ELEMS // 16, zstep, 0)

    for p in range(N_PASSES):
        r0 = SC_ROW0 + (cid * N_PASSES + p) * ROWS_PER_PASS
        pltpu.sync_copy(zb, shared.at[pl.ds(sid * SUB_ELEMS, SUB_ELEMS)])
        plsc.subcore_barrier()

        def estep(i, carry):
            d = dstb[pl.ds(i * 16, 16)]
            s = srcb[pl.ds(i * 16, 16)]
            inr = (d >= r0) & (d < r0 + ROWS_PER_PASS)
            flat = (d - r0) * N_NODES + s
            row = i // 8
            col = (i % 8) * 16
            idxb[row, pl.ds(col, 16)] = jnp.where(inr, flat, 0)
            valb[row, pl.ds(col, 16)] = jnp.where(inr, 1.0, 0.0)
            return carry

        lax.fori_loop(0, E_VREGS, estep, 0)
        for j in range(IDX_ROWS):
            pltpu.sync_copy(valb.at[j], shared.at[idxb.at[j]], add=True)
        plsc.subcore_barrier()
        out_base = (r0 - SC_ROW0) * N_NODES + sid * SUB_ELEMS
        pltpu.sync_copy(shared.at[pl.ds(sid * SUB_ELEMS, SUB_ELEMS)],
                        out_hbm.at[pl.ds(out_base, SUB_ELEMS)])


def _sc_counts(dst, src):
    mesh = plsc.VectorSubcoreMesh(
        core_axis_name="c", subcore_axis_name="s", num_cores=2)
    fn = functools.partial(
        pl.kernel,
        mesh=mesh,
        out_type=jax.ShapeDtypeStruct((SC_ROWS * N_NODES,), jnp.float32),
        scratch_types=[
            pltpu.VMEM((E_PER_SUB,), jnp.int32),
            pltpu.VMEM((E_PER_SUB,), jnp.int32),
            pltpu.VMEM((IDX_ROWS, 128), jnp.int32),
            pltpu.VMEM((IDX_ROWS, 128), jnp.float32),
            pltpu.VMEM((SUB_ELEMS,), jnp.float32),
            pltpu.VMEM_SHARED((PASS_ELEMS,), jnp.float32),
        ],
    )(_sc_counts_body)
    return fn(dst, src)


def _top_body(dst_ref, src_ref, c_ref):
    i = pl.program_id(0)

    @pl.when(i == 0)
    def _():
        c_ref[...] = jnp.zeros_like(c_ref)

    d = dst_ref[0]
    s = src_ref[0]
    iota_r = jax.lax.broadcasted_iota(jnp.int32, (E_CHUNK, SC_ROW0), 1)
    iota_n = jax.lax.broadcasted_iota(jnp.int32, (E_CHUNK, N_NODES), 1)
    D = (jnp.broadcast_to(d, (E_CHUNK, SC_ROW0)) == iota_r).astype(jnp.bfloat16)
    S = (jnp.broadcast_to(s, (E_CHUNK, N_NODES)) == iota_n).astype(jnp.bfloat16)
    c_ref[...] += jax.lax.dot_general(
        D, S, (((0,), (0,)), ((), ())), preferred_element_type=jnp.float32)


def _top_counts(dst, src):
    return pl.pallas_call(
        _top_body,
        grid=(N_ECHUNK,),
        in_specs=[
            pl.BlockSpec((1, E_CHUNK, 1), lambda i: (i, 0, 0)),
            pl.BlockSpec((1, E_CHUNK, 1), lambda i: (i, 0, 0)),
        ],
        out_specs=pl.BlockSpec((SC_ROW0, N_NODES), lambda i: (0, 0)),
        out_shape=jax.ShapeDtypeStruct((SC_ROW0, N_NODES), jnp.float32),
    )(dst.reshape(N_ECHUNK, E_CHUNK, 1), src.reshape(N_ECHUNK, E_CHUNK, 1))


def _encode_body(x_ref, y_ref, pos_ref, w1_ref, b1_ref, w2_ref, b2_ref, g_ref):
    x = x_ref[0]
    y = y_ref[0]
    xin = jnp.concatenate([x, y], axis=1)
    h = jnp.maximum(
        jnp.dot(xin, w1_ref[...], preferred_element_type=jnp.float32)
        + b1_ref[...], 0.0)
    feats = jnp.dot(h, w2_ref[...], preferred_element_type=jnp.float32) + b2_ref[...]
    p = pos_ref[...]
    xp = jax.lax.dot_general(x, p, (((1,), (1,)), ((), ())),
                             preferred_element_type=jnp.float32)
    x2 = jnp.sum(x * x, axis=1, keepdims=True)
    p2 = jnp.transpose(jnp.sum(p * p, axis=1, keepdims=True))
    logits = 2.0 * xp - x2 - p2
    m = jnp.max(logits, axis=1, keepdims=True)
    e = jnp.exp(logits - m)
    coord = e / jnp.sum(e, axis=1, keepdims=True)
    g_ref[0] = jax.lax.dot_general(
        coord, feats, (((0,), (0,)), ((), ())), preferred_element_type=jnp.float32)


def _encode(inp_x, inp_y, pos, p):
    return pl.pallas_call(
        _encode_body,
        grid=(BS,),
        in_specs=[
            pl.BlockSpec((1, 1024, 2), lambda b: (b, 0, 0)),
            pl.BlockSpec((1, 1024, 1), lambda b: (b, 0, 0)),
            pl.BlockSpec((N_NODES, 2), lambda b: (0, 0)),
            pl.BlockSpec((3, 64), lambda b: (0, 0)),
            pl.BlockSpec((1, 64), lambda b: (0, 0)),
            pl.BlockSpec((64, F), lambda b: (0, 0)),
            pl.BlockSpec((1, F), lambda b: (0, 0)),
        ],
        out_specs=pl.BlockSpec((1, N_NODES, F), lambda b: (b, 0, 0)),
        out_shape=jax.ShapeDtypeStruct((BS, N_NODES, F), jnp.float32),
    )(inp_x, inp_y, pos,
      p["enc1"]["W"], p["enc1"]["b"][None, :],
      p["enc2"]["W"], p["enc2"]["b"][None, :])


def _mp_body(ct_ref, cb_ref, x0_ref, pos_ref, wp_ref, wb_ref, bc_ref, gc_ref,
             bec_ref, mavg_ref, out_ref, x_s, k_s, a_s):
    Ct = ct_ref[...]
    Cb = cb_ref[...]
    deg = jnp.concatenate(
        [jnp.sum(Ct, axis=1, keepdims=True),
         jnp.sum(Cb, axis=1, keepdims=True)], axis=0)
    dinv = jax.lax.rsqrt(deg)
    dinv_t = jnp.transpose(dinv)
    a_s[pl.ds(0, SC_ROW0), :] = Ct * dinv[:SC_ROW0] * dinv_t
    a_s[pl.ds(SC_ROW0, SC_ROWS), :] = Cb * dinv[SC_ROW0:] * dinv_t
    pw = jnp.dot(pos_ref[...], wp_ref[...], preferred_element_type=jnp.float32)
    k_s[...] = jnp.dot(a_s[...], pw, preferred_element_type=jnp.float32) + bc_ref[...]
    x_s[...] = x0_ref[...]
    gc = gc_ref[...]
    bec = bec_ref[...]

    def step(i, carry):
        X = x_s[...]
        T = jnp.dot(X, wb_ref[...], preferred_element_type=jnp.float32)
        Y = X + jnp.dot(a_s[...], T, preferred_element_type=jnp.float32) + k_s[...]
        M = jnp.dot(Y, mavg_ref[...], preferred_element_type=jnp.float32)
        Yc = Y - M
        V = jnp.dot(Yc * Yc, mavg_ref[...], preferred_element_type=jnp.float32)
        x_s[...] = Yc * jax.lax.rsqrt(V + 1e-5) * gc + bec
        return carry

    jax.lax.fori_loop(0, MSG_STEPS, step, 0)
    out_ref[...] = x_s[...]


def _message_pass(Ct, Cb, X0, pos, p):
    wx_g = p["conv"]["W"][2:, :]
    wx_v = p["conv1"]["W"][2:, :]
    eye4 = jnp.eye(4, dtype=jnp.float32)
    wbig = jnp.zeros((256, 256), jnp.float32)
    wbig = wbig.at[:128, :128].set(jnp.kron(eye4, wx_g))
    wbig = wbig.at[128:, 128:].set(jnp.kron(eye4, wx_v))
    wpcat = jnp.concatenate(
        [jnp.tile(p["conv"]["W"][:2, :], (1, 4)),
         jnp.tile(p["conv1"]["W"][:2, :], (1, 4))], axis=1)
    bcat = jnp.concatenate(
        [jnp.tile(p["conv"]["b"], 4), jnp.tile(p["conv1"]["b"], 4)])[None, :]
    gcat = jnp.concatenate(
        [jnp.tile(p["ln_g"], 4), jnp.tile(p["ln1_g"], 4)])[None, :]
    becat = jnp.concatenate(
        [jnp.tile(p["ln_b"], 4), jnp.tile(p["ln1_b"], 4)])[None, :]
    mavg = jnp.kron(jnp.eye(8, dtype=jnp.float32),
                    jnp.full((F, F), 1.0 / F, jnp.float32))
    return pl.pallas_call(
        _mp_body,
        in_specs=[
            pl.BlockSpec((SC_ROW0, N_NODES), lambda: (0, 0)),
            pl.BlockSpec((SC_ROWS, N_NODES), lambda: (0, 0)),
            pl.BlockSpec((N_NODES, 256), lambda: (0, 0)),
            pl.BlockSpec((N_NODES, 2), lambda: (0, 0)),
            pl.BlockSpec((2, 256), lambda: (0, 0)),
            pl.BlockSpec((256, 256), lambda: (0, 0)),
            pl.BlockSpec((1, 256), lambda: (0, 0)),
            pl.BlockSpec((1, 256), lambda: (0, 0)),
            pl.BlockSpec((1, 256), lambda: (0, 0)),
            pl.BlockSpec((256, 256), lambda: (0, 0)),
        ],
        out_specs=pl.BlockSpec((N_NODES, 256), lambda: (0, 0)),
        out_shape=jax.ShapeDtypeStruct((N_NODES, 256), jnp.float32),
        scratch_shapes=[
            pltpu.VMEM((N_NODES, 256), jnp.float32),
            pltpu.VMEM((N_NODES, 256), jnp.float32),
            pltpu.VMEM((N_NODES, N_NODES), jnp.float32),
        ],
    )(Ct, Cb, X0, pos, wpcat, wbig, bcat, gcat, becat, mavg)


def _attn_body(x_ref, wq_ref, wk_ref, wv_ref, wo_ref, out_ref):
    x = x_ref[0]
    q = jnp.dot(x, wq_ref[0], preferred_element_type=jnp.float32)
    k = jnp.dot(x, wk_ref[0], preferred_element_type=jnp.float32)
    v = jnp.dot(x, wv_ref[0], preferred_element_type=jnp.float32)
    s = jax.lax.dot_general(q, k, (((1,), (1,)), ((), ())),
                            preferred_element_type=jnp.float32)
    s = s * (1.0 / np.sqrt(F))
    m = jnp.max(s, axis=1, keepdims=True)
    e = jnp.exp(s - m)
    a = e / jnp.sum(e, axis=1, keepdims=True)
    out_ref[0] = jnp.dot(
        jnp.dot(a, v, preferred_element_type=jnp.float32), wo_ref[0],
        preferred_element_type=jnp.float32)


def _attention(X, p):
    wq = jnp.stack([p["attn"]["Wq"]] * 4 + [p["attn1"]["Wq"]] * 4)
    wk = jnp.stack([p["attn"]["Wk"]] * 4 + [p["attn1"]["Wk"]] * 4)
    wv = jnp.stack([p["attn"]["Wv"]] * 4 + [p["attn1"]["Wv"]] * 4)
    wo = jnp.stack([p["attn"]["Wo"]] * 4 + [p["attn1"]["Wo"]] * 4)
    return pl.pallas_call(
        _attn_body,
        grid=(8,),
        in_specs=[
            pl.BlockSpec((1, N_NODES, F), lambda c: (c, 0, 0)),
            pl.BlockSpec((1, F, F), lambda c: (c, 0, 0)),
            pl.BlockSpec((1, F, F), lambda c: (c, 0, 0)),
            pl.BlockSpec((1, F, F), lambda c: (c, 0, 0)),
            pl.BlockSpec((1, F, F), lambda c: (c, 0, 0)),
        ],
        out_specs=pl.BlockSpec((1, N_NODES, F), lambda c: (c, 0, 0)),
        out_shape=jax.ShapeDtypeStruct((8, N_NODES, F), jnp.float32),
    )(X, wq, wk, wv, wo)


def _readout_body(q_ref, pos_ref, gx_ref, xv_ref,
                  t1w_ref, t1b_ref, t2w_ref, t2b_ref,
                  v1w_ref, v1b_ref, v2w_ref, v2b_ref,
                  lng_ref, lnb_ref, v3w_ref, v3b_ref, v4w_ref, v4b_ref,
                  out_ref):
    qb = q_ref[0]
    ppos = pos_ref[...]
    qp = jax.lax.dot_general(qb, ppos, (((1,), (1,)), ((), ())),
                             preferred_element_type=jnp.float32)
    q2 = jnp.sum(qb * qb, axis=1, keepdims=True)
    p2 = jnp.transpose(jnp.sum(ppos * ppos, axis=1, keepdims=True))
    logits = 2.0 * qp - q2 - p2
    m = jnp.max(logits, axis=1, keepdims=True)
    e = jnp.exp(logits - m)
    coord = e / jnp.sum(e, axis=1, keepdims=True)
    lat1 = jnp.dot(coord, gx_ref[0], preferred_element_type=jnp.float32)
    lat2 = jnp.dot(coord, xv_ref[0], preferred_element_type=jnp.float32)

    tin = jnp.concatenate([lat1, qb], axis=1)
    t = jnp.maximum(jnp.dot(tin, t1w_ref[...], preferred_element_type=jnp.float32)
                    + t1b_ref[...], 0.0)
    t = jnp.dot(t, t2w_ref[...], preferred_element_type=jnp.float32) + t2b_ref[...]

    vin = jnp.concatenate([lat2, qb], axis=1)
    h = jnp.maximum(jnp.dot(vin, v1w_ref[...], preferred_element_type=jnp.float32)
                    + v1b_ref[...], 0.0)
    h = jnp.dot(h, v2w_ref[...], preferred_element_type=jnp.float32) + v2b_ref[...]
    hm = jnp.mean(h, axis=1, keepdims=True)
    hc = h - hm
    hv = jnp.mean(hc * hc, axis=1, keepdims=True)
    h = hc * jax.lax.rsqrt(hv + 1e-5) * lng_ref[...] + lnb_ref[...]
    h = jnp.maximum(h, 0.0)
    h = jnp.maximum(jnp.dot(h, v3w_ref[...], preferred_element_type=jnp.float32)
                    + v3b_ref[...], 0.0)
    v = jnp.maximum(jnp.dot(h, v4w_ref[...], preferred_element_type=jnp.float32)
                    + v4b_ref[...], 0.0)
    out_ref[0] = jnp.concatenate([t, v], axis=1)


def _readout(q, pos, Xa, p):
    full = lambda shape: pl.BlockSpec(shape, lambda b: tuple(0 for _ in shape))
    return pl.pallas_call(
        _readout_body,
        grid=(BS,),
        in_specs=[
            pl.BlockSpec((1, 1024, 2), lambda b: (b, 0, 0)),
            pl.BlockSpec((N_NODES, 2), lambda b: (0, 0)),
            pl.BlockSpec((1, N_NODES, F), lambda b: (b, 0, 0)),
            pl.BlockSpec((1, N_NODES, F), lambda b: (4 + b, 0, 0)),
            full((34, 64)), full((1, 64)), full((64, 1)), full((1, 1)),
            full((34, 32)), full((1, 32)), full((32, 32)), full((1, 32)),
            full((1, 32)), full((1, 32)),
            full((32, 16)), full((1, 16)), full((16, 1)), full((1, 1)),
        ],
        out_specs=pl.BlockSpec((1, 1024, 2), lambda b: (b, 0, 0)),
        out_shape=jax.ShapeDtypeStruct((BS, 1024, 2), jnp.float32),
    )(q, pos, Xa, Xa,
      p["dect1"]["W"], p["dect1"]["b"][None, :],
      p["dect2"]["W"], p["dect2"]["b"][None, :],
      p["decv1"]["W"], p["decv1"]["b"][None, :],
      p["decv2"]["W"], p["decv2"]["b"][None, :],
      p["decv_ln_g"][None, :], p["decv_ln_b"][None, :],
      p["decv3"]["W"], p["decv3"]["b"][None, :],
      p["decv4"]["W"], p["decv4"]["b"][None, :])


def kernel(inp_x, inp_y, q, pos, edge_index, params):
    loop = jnp.arange(N_NODES, dtype=edge_index.dtype)
    src = jnp.concatenate([edge_index[0], loop])
    dst = jnp.concatenate([edge_index[1], loop])
    Cb = _sc_counts(dst, src).reshape(SC_ROWS, N_NODES)
    Ct = _top_counts(dst, src)
    G = _encode(inp_x, inp_y, pos, params)
    Gf = jnp.transpose(G, (1, 0, 2)).reshape(N_NODES, BS * F)
    X0 = jnp.concatenate([Gf, Gf], axis=1)
    X = _message_pass(Ct, Cb, X0, pos, params)
    Xc = jnp.transpose(X.reshape(N_NODES, 8, F), (1, 0, 2))
    Xa = _attention(Xc, params)
    return _readout(q, pos, Xa, params)


# speedup vs baseline: 1.2020x; 1.2020x over previous
import functools
import jax
import jax.numpy as jnp
import numpy as np
from jax import lax
from jax.experimental import pallas as pl
from jax.experimental.pallas import tpu as pltpu
from jax.experimental.pallas import tpu_sc as plsc

N_NODES = 2048
N_EDGES = 32768
F = 32
BS = 4
MSG_STEPS = 20
E_TOT = N_EDGES + N_NODES
E_CHUNK = 1024
N_ECHUNK = E_TOT // E_CHUNK


E_PER_SUB = E_TOT // 16
E_VREGS = E_PER_SUB // 16
IDX_ROWS = E_PER_SUB // 128
ROWS_PER_PASS = 256
N_PASSES = N_NODES // 2 // ROWS_PER_PASS
PASS_ELEMS = ROWS_PER_PASS * N_NODES
SUB_ELEMS = PASS_ELEMS // 16


def _sc_counts_body(dst_hbm, src_hbm, out_hbm, dstb, srcb, idxb, valb, zb,
                    shared):
    cid = lax.axis_index("c")
    sid = lax.axis_index("s")
    base = sid * E_PER_SUB
    pltpu.sync_copy(dst_hbm.at[pl.ds(base, E_PER_SUB)], dstb)
    pltpu.sync_copy(src_hbm.at[pl.ds(base, E_PER_SUB)], srcb)

    def zstep(i, carry):
        zb[pl.ds(i * 16, 16)] = jnp.zeros((16,), jnp.float32)
        return carry

    lax.fori_loop(0, SUB_ELEMS // 16, zstep, 0)

    for p in range(N_PASSES):
        r0 = (cid * N_PASSES + p) * ROWS_PER_PASS
        pltpu.sync_copy(zb, shared.at[pl.ds(sid * SUB_ELEMS, SUB_ELEMS)])
        plsc.subcore_barrier()

        def estep(i, carry):
            d = dstb[pl.ds(i * 16, 16)]
            s = srcb[pl.ds(i * 16, 16)]
            inr = (d >= r0) & (d < r0 + ROWS_PER_PASS)
            flat = (d - r0) * N_NODES + s
            row = i // 8
            col = (i % 8) * 16
            idxb[row, pl.ds(col, 16)] = jnp.where(inr, flat, 0)
            valb[row, pl.ds(col, 16)] = jnp.where(inr, 1.0, 0.0)
            return carry

        lax.fori_loop(0, E_VREGS, estep, 0)
        for j in range(IDX_ROWS):
            pltpu.sync_copy(valb.at[j], shared.at[idxb.at[j]], add=True)
        plsc.subcore_barrier()
        out_base = r0 * N_NODES + sid * SUB_ELEMS
        pltpu.sync_copy(shared.at[pl.ds(sid * SUB_ELEMS, SUB_ELEMS)],
                        out_hbm.at[pl.ds(out_base, SUB_ELEMS)])


def _sc_counts(dst, src):
    mesh = plsc.VectorSubcoreMesh(
        core_axis_name="c", subcore_axis_name="s", num_cores=2)
    fn = functools.partial(
        pl.kernel,
        mesh=mesh,
        out_type=jax.ShapeDtypeStruct((N_NODES * N_NODES,), jnp.float32),
        scratch_types=[
            pltpu.VMEM((E_PER_SUB,), jnp.int32),
            pltpu.VMEM((E_PER_SUB,), jnp.int32),
            pltpu.VMEM((IDX_ROWS, 128), jnp.int32),
            pltpu.VMEM((IDX_ROWS, 128), jnp.float32),
            pltpu.VMEM((SUB_ELEMS,), jnp.float32),
            pltpu.VMEM_SHARED((PASS_ELEMS,), jnp.float32),
        ],
    )(_sc_counts_body)
    return fn(dst, src)


def _encode_body(x_ref, y_ref, pos_ref, w1_ref, b1_ref, w2_ref, b2_ref, g_ref):
    x = x_ref[0]
    y = y_ref[0]
    xin = jnp.concatenate([x, y], axis=1)
    h = jnp.maximum(
        jnp.dot(xin, w1_ref[...], preferred_element_type=jnp.float32)
        + b1_ref[...], 0.0)
    feats = jnp.dot(h, w2_ref[...], preferred_element_type=jnp.float32) + b2_ref[...]
    p = pos_ref[...]
    xp = jax.lax.dot_general(x, p, (((1,), (1,)), ((), ())),
                             preferred_element_type=jnp.float32)
    x2 = jnp.sum(x * x, axis=1, keepdims=True)
    p2 = jnp.transpose(jnp.sum(p * p, axis=1, keepdims=True))
    logits = 2.0 * xp - x2 - p2
    m = jnp.max(logits, axis=1, keepdims=True)
    e = jnp.exp(logits - m)
    coord = e / jnp.sum(e, axis=1, keepdims=True)
    g_ref[0] = jax.lax.dot_general(
        coord, feats, (((0,), (0,)), ((), ())), preferred_element_type=jnp.float32)


def _encode(inp_x, inp_y, pos, p):
    return pl.pallas_call(
        _encode_body,
        grid=(BS,),
        in_specs=[
            pl.BlockSpec((1, 1024, 2), lambda b: (b, 0, 0)),
            pl.BlockSpec((1, 1024, 1), lambda b: (b, 0, 0)),
            pl.BlockSpec((N_NODES, 2), lambda b: (0, 0)),
            pl.BlockSpec((3, 64), lambda b: (0, 0)),
            pl.BlockSpec((1, 64), lambda b: (0, 0)),
            pl.BlockSpec((64, F), lambda b: (0, 0)),
            pl.BlockSpec((1, F), lambda b: (0, 0)),
        ],
        out_specs=pl.BlockSpec((1, N_NODES, F), lambda b: (b, 0, 0)),
        out_shape=jax.ShapeDtypeStruct((BS, N_NODES, F), jnp.float32),
    )(inp_x, inp_y, pos,
      p["enc1"]["W"], p["enc1"]["b"][None, :],
      p["enc2"]["W"], p["enc2"]["b"][None, :])


def _mp_body(c_ref, x0_ref, pos_ref, wp_ref, wb_ref, bc_ref, gc_ref,
             bec_ref, mavg_ref, out_ref, x_s, k_s, a_s):
    C = c_ref[...]
    deg = jnp.sum(C, axis=1, keepdims=True)
    dinv = jax.lax.rsqrt(deg)
    a_s[...] = C * dinv * jnp.transpose(dinv)
    pw = jnp.dot(pos_ref[...], wp_ref[...], preferred_element_type=jnp.float32)
    k_s[...] = jnp.dot(a_s[...], pw, preferred_element_type=jnp.float32) + bc_ref[...]
    x_s[...] = x0_ref[...]
    gc = gc_ref[...]
    bec = bec_ref[...]

    def step(i, carry):
        X = x_s[...]
        T = jnp.dot(X, wb_ref[...], preferred_element_type=jnp.float32)
        Y = X + jnp.dot(a_s[...], T, preferred_element_type=jnp.float32) + k_s[...]
        M = jnp.dot(Y, mavg_ref[...], preferred_element_type=jnp.float32)
        Yc = Y - M
        V = jnp.dot(Yc * Yc, mavg_ref[...], preferred_element_type=jnp.float32)
        x_s[...] = Yc * jax.lax.rsqrt(V + 1e-5) * gc + bec
        return carry

    jax.lax.fori_loop(0, MSG_STEPS, step, 0)
    out_ref[...] = x_s[...]


def _message_pass(C, X0, pos, p):
    wx_g = p["conv"]["W"][2:, :]
    wx_v = p["conv1"]["W"][2:, :]
    eye4 = jnp.eye(4, dtype=jnp.float32)
    wbig = jnp.zeros((256, 256), jnp.float32)
    wbig = wbig.at[:128, :128].set(jnp.kron(eye4, wx_g))
    wbig = wbig.at[128:, 128:].set(jnp.kron(eye4, wx_v))
    wpcat = jnp.concatenate(
        [jnp.tile(p["conv"]["W"][:2, :], (1, 4)),
         jnp.tile(p["conv1"]["W"][:2, :], (1, 4))], axis=1)
    bcat = jnp.concatenate(
        [jnp.tile(p["conv"]["b"], 4), jnp.tile(p["conv1"]["b"], 4)])[None, :]
    gcat = jnp.concatenate(
        [jnp.tile(p["ln_g"], 4), jnp.tile(p["ln1_g"], 4)])[None, :]
    becat = jnp.concatenate(
        [jnp.tile(p["ln_b"], 4), jnp.tile(p["ln1_b"], 4)])[None, :]
    mavg = jnp.kron(jnp.eye(8, dtype=jnp.float32),
                    jnp.full((F, F), 1.0 / F, jnp.float32))
    return pl.pallas_call(
        _mp_body,
        in_specs=[
            pl.BlockSpec((N_NODES, N_NODES), lambda: (0, 0)),
            pl.BlockSpec((N_NODES, 256), lambda: (0, 0)),
            pl.BlockSpec((N_NODES, 2), lambda: (0, 0)),
            pl.BlockSpec((2, 256), lambda: (0, 0)),
            pl.BlockSpec((256, 256), lambda: (0, 0)),
            pl.BlockSpec((1, 256), lambda: (0, 0)),
            pl.BlockSpec((1, 256), lambda: (0, 0)),
            pl.BlockSpec((1, 256), lambda: (0, 0)),
            pl.BlockSpec((256, 256), lambda: (0, 0)),
        ],
        out_specs=pl.BlockSpec((N_NODES, 256), lambda: (0, 0)),
        out_shape=jax.ShapeDtypeStruct((N_NODES, 256), jnp.float32),
        scratch_shapes=[
            pltpu.VMEM((N_NODES, 256), jnp.float32),
            pltpu.VMEM((N_NODES, 256), jnp.float32),
            pltpu.VMEM((N_NODES, N_NODES), jnp.float32),
        ],
    )(C, X0, pos, wpcat, wbig, bcat, gcat, becat, mavg)


def _attn_body(x_ref, wq_ref, wk_ref, wv_ref, wo_ref, out_ref):
    x = x_ref[0]
    q = jnp.dot(x, wq_ref[0], preferred_element_type=jnp.float32)
    k = jnp.dot(x, wk_ref[0], preferred_element_type=jnp.float32)
    v = jnp.dot(x, wv_ref[0], preferred_element_type=jnp.float32)
    s = jax.lax.dot_general(q, k, (((1,), (1,)), ((), ())),
                            preferred_element_type=jnp.float32)
    s = s * (1.0 / np.sqrt(F))
    m = jnp.max(s, axis=1, keepdims=True)
    e = jnp.exp(s - m)
    a = e / jnp.sum(e, axis=1, keepdims=True)
    out_ref[0] = jnp.dot(
        jnp.dot(a, v, preferred_element_type=jnp.float32), wo_ref[0],
        preferred_element_type=jnp.float32)


def _attention(X, p):
    wq = jnp.stack([p["attn"]["Wq"]] * 4 + [p["attn1"]["Wq"]] * 4)
    wk = jnp.stack([p["attn"]["Wk"]] * 4 + [p["attn1"]["Wk"]] * 4)
    wv = jnp.stack([p["attn"]["Wv"]] * 4 + [p["attn1"]["Wv"]] * 4)
    wo = jnp.stack([p["attn"]["Wo"]] * 4 + [p["attn1"]["Wo"]] * 4)
    return pl.pallas_call(
        _attn_body,
        grid=(8,),
        in_specs=[
            pl.BlockSpec((1, N_NODES, F), lambda c: (c, 0, 0)),
            pl.BlockSpec((1, F, F), lambda c: (c, 0, 0)),
            pl.BlockSpec((1, F, F), lambda c: (c, 0, 0)),
            pl.BlockSpec((1, F, F), lambda c: (c, 0, 0)),
            pl.BlockSpec((1, F, F), lambda c: (c, 0, 0)),
        ],
        out_specs=pl.BlockSpec((1, N_NODES, F), lambda c: (c, 0, 0)),
        out_shape=jax.ShapeDtypeStruct((8, N_NODES, F), jnp.float32),
    )(X, wq, wk, wv, wo)


def _readout_body(q_ref, pos_ref, gx_ref, xv_ref,
                  t1w_ref, t1b_ref, t2w_ref, t2b_ref,
                  v1w_ref, v1b_ref, v2w_ref, v2b_ref,
                  lng_ref, lnb_ref, v3w_ref, v3b_ref, v4w_ref, v4b_ref,
                  out_ref):
    qb = q_ref[0]
    ppos = pos_ref[...]
    qp = jax.lax.dot_general(qb, ppos, (((1,), (1,)), ((), ())),
                             preferred_element_type=jnp.float32)
    q2 = jnp.sum(qb * qb, axis=1, keepdims=True)
    p2 = jnp.transpose(jnp.sum(ppos * ppos, axis=1, keepdims=True))
    logits = 2.0 * qp - q2 - p2
    m = jnp.max(logits, axis=1, keepdims=True)
    e = jnp.exp(logits - m)
    coord = e / jnp.sum(e, axis=1, keepdims=True)
    lat1 = jnp.dot(coord, gx_ref[0], preferred_element_type=jnp.float32)
    lat2 = jnp.dot(coord, xv_ref[0], preferred_element_type=jnp.float32)

    tin = jnp.concatenate([lat1, qb], axis=1)
    t = jnp.maximum(jnp.dot(tin, t1w_ref[...], preferred_element_type=jnp.float32)
                    + t1b_ref[...], 0.0)
    t = jnp.dot(t, t2w_ref[...], preferred_element_type=jnp.float32) + t2b_ref[...]

    vin = jnp.concatenate([lat2, qb], axis=1)
    h = jnp.maximum(jnp.dot(vin, v1w_ref[...], preferred_element_type=jnp.float32)
                    + v1b_ref[...], 0.0)
    h = jnp.dot(h, v2w_ref[...], preferred_element_type=jnp.float32) + v2b_ref[...]
    hm = jnp.mean(h, axis=1, keepdims=True)
    hc = h - hm
    hv = jnp.mean(hc * hc, axis=1, keepdims=True)
    h = hc * jax.lax.rsqrt(hv + 1e-5) * lng_ref[...] + lnb_ref[...]
    h = jnp.maximum(h, 0.0)
    h = jnp.maximum(jnp.dot(h, v3w_ref[...], preferred_element_type=jnp.float32)
                    + v3b_ref[...], 0.0)
    v = jnp.maximum(jnp.dot(h, v4w_ref[...], preferred_element_type=jnp.float32)
                    + v4b_ref[...], 0.0)
    out_ref[0] = jnp.concatenate([t, v], axis=1)


def _readout(q, pos, Xa, p):
    full = lambda shape: pl.BlockSpec(shape, lambda b: tuple(0 for _ in shape))
    return pl.pallas_call(
        _readout_body,
        grid=(BS,),
        in_specs=[
            pl.BlockSpec((1, 1024, 2), lambda b: (b, 0, 0)),
            pl.BlockSpec((N_NODES, 2), lambda b: (0, 0)),
            pl.BlockSpec((1, N_NODES, F), lambda b: (b, 0, 0)),
            pl.BlockSpec((1, N_NODES, F), lambda b: (4 + b, 0, 0)),
            full((34, 64)), full((1, 64)), full((64, 1)), full((1, 1)),
            full((34, 32)), full((1, 32)), full((32, 32)), full((1, 32)),
            full((1, 32)), full((1, 32)),
            full((32, 16)), full((1, 16)), full((16, 1)), full((1, 1)),
        ],
        out_specs=pl.BlockSpec((1, 1024, 2), lambda b: (b, 0, 0)),
        out_shape=jax.ShapeDtypeStruct((BS, 1024, 2), jnp.float32),
    )(q, pos, Xa, Xa,
      p["dect1"]["W"], p["dect1"]["b"][None, :],
      p["dect2"]["W"], p["dect2"]["b"][None, :],
      p["decv1"]["W"], p["decv1"]["b"][None, :],
      p["decv2"]["W"], p["decv2"]["b"][None, :],
      p["decv_ln_g"][None, :], p["decv_ln_b"][None, :],
      p["decv3"]["W"], p["decv3"]["b"][None, :],
      p["decv4"]["W"], p["decv4"]["b"][None, :])


def kernel(inp_x, inp_y, q, pos, edge_index, params):
    loop = jnp.arange(N_NODES, dtype=edge_index.dtype)
    src = jnp.concatenate([edge_index[0], loop])
    dst = jnp.concatenate([edge_index[1], loop])
    C = _sc_counts(dst, src).reshape(N_NODES, N_NODES)
    G = _encode(inp_x, inp_y, pos, params)
    Gf = jnp.transpose(G, (1, 0, 2)).reshape(N_NODES, BS * F)
    X0 = jnp.concatenate([Gf, Gf], axis=1)
    X = _message_pass(C, X0, pos, params)
    Xc = jnp.transpose(X.reshape(N_NODES, 8, F), (1, 0, 2))
    Xa = _attention(Xc, params)
    return _readout(q, pos, Xa, params)
